# baseline (speedup 1.0000x reference)
import functools

import jax
import jax.numpy as jnp
from jax import lax
from jax.experimental import pallas as pl
from jax.experimental.pallas import tpu as pltpu
from jax.experimental.pallas import tpu_sc as plsc

N_UNITS = 1000
K_SUB = 5
BATCH = 1024
N_RECEPTORS = 4096

NC = 2
NS = 16
NW = NC * NS
LANES = 16
B_PER_W = BATCH // NW
B_HALF = B_PER_W // 2
CHUNKS = N_RECEPTORS // LANES
W_INT = 2 * N_UNITS
W_PAD = 2048


def _stage1_body(x_ref, c_ref, a_ref):
    x = x_ref[...]
    lnc = jnp.log(c_ref[...] + 1e-12)
    even = (
        lax.broadcasted_iota(jnp.int32, x.shape, 1) % 2
    ) == 0
    xm = lnc - x
    sp = jnp.maximum(xm, 0.0) + jnp.log1p(jnp.exp(-jnp.abs(xm)))
    g = jnp.where(even, x - lnc, sp)
    a_ref[:, :W_INT] = g + jnp.roll(g, -1, axis=1)


def _stage1(x, conc2d):
    bb = 256
    return pl.pallas_call(
        _stage1_body,
        grid=(BATCH // bb,),
        in_specs=[
            pl.BlockSpec((bb, W_INT), lambda i: (i, 0)),
            pl.BlockSpec((bb, 1), lambda i: (i, 0)),
        ],
        out_specs=pl.BlockSpec((bb, W_PAD), lambda i: (i, 0)),
        out_shape=jax.ShapeDtypeStruct((BATCH, W_PAD), jnp.float32),
    )(x, conc2d)


_SC_MESH = plsc.VectorSubcoreMesh(
    core_axis_name="c", subcore_axis_name="s", num_cores=NC, num_subcores=NS
)


@functools.partial(
    pl.kernel,
    mesh=_SC_MESH,
    out_type=jax.ShapeDtypeStruct((BATCH, N_RECEPTORS), jnp.float32),
    compiler_params=pltpu.CompilerParams(needs_layout_passes=False),
    scratch_types=[
        pltpu.VMEM((N_RECEPTORS * K_SUB,), jnp.int32),
        pltpu.VMEM((B_HALF, W_PAD), jnp.float32),
        pltpu.VMEM((B_HALF, N_RECEPTORS), jnp.float32),
    ],
)
def _sc_gather_sigmoid(a_hbm, idx_hbm, out_hbm, idx_v, a_v, out_v):
    wid = lax.axis_index("s") * NC + lax.axis_index("c")
    pltpu.sync_copy(idx_hbm, idx_v)
    iota5 = lax.iota(jnp.int32, LANES) * K_SUB

    for half in range(2):
        base = wid * B_PER_W + half * B_HALF
        pltpu.sync_copy(a_hbm.at[pl.ds(base, B_HALF)], a_v)

        @pl.loop(0, CHUNKS, unroll=2)
        def _chunk_loop(ch):
            off = ch * LANES
            iks = [
                2 * plsc.load_gather(idx_v, [iota5 + (off * K_SUB + k)])
                for k in range(K_SUB)
            ]

            res = []
            for j in range(B_HALF):
                jv = jnp.full((LANES,), j, dtype=jnp.int32)
                g = [plsc.load_gather(a_v, [jv, iks[k]]) for k in range(K_SUB)]
                s = (g[0] + g[1]) + (g[2] + g[3]) + g[4]
                res.append(1.0 / (1.0 + jnp.exp(s)))
            for j in range(B_HALF):
                out_v[j, pl.ds(off, LANES)] = res[j]

        pltpu.sync_copy(out_v, out_hbm.at[pl.ds(wid * B_PER_W + half * B_HALF, B_HALF)])


def kernel(energies, concentrations, receptor_indices):
    x = energies.reshape(BATCH, W_INT)
    conc2d = concentrations.reshape(BATCH, 1)
    a = _stage1(x, conc2d)
    idx_flat = receptor_indices.astype(jnp.int32).reshape(-1)
    return _sc_gather_sigmoid(a, idx_flat)


# speedup vs baseline: 1.0017x; 1.0017x over previous
import functools

import jax
import jax.numpy as jnp
from jax import lax
from jax.experimental import pallas as pl
from jax.experimental.pallas import tpu as pltpu
from jax.experimental.pallas import tpu_sc as plsc

N_UNITS = 1000
K_SUB = 5
BATCH = 1024
N_RECEPTORS = 4096

NC = 2
NS = 16
NW = NC * NS
LANES = 16
B_PER_W = BATCH // NW
B_HALF = B_PER_W // 2
CHUNKS = N_RECEPTORS // LANES
W_INT = 2 * N_UNITS
W_PAD = 2048


def _stage1_body(x_ref, c_ref, a_ref):
    x = x_ref[...]
    lnc = jnp.log(c_ref[...] + 1e-12)
    even = (
        lax.broadcasted_iota(jnp.int32, x.shape, 1) % 2
    ) == 0
    xm = lnc - x
    sp = jnp.maximum(xm, 0.0) + jnp.log1p(jnp.exp(-jnp.abs(xm)))
    g = jnp.where(even, x - lnc, sp)
    a_ref[:, :W_INT] = g + jnp.roll(g, -1, axis=1)


def _stage1(x, conc2d):
    bb = 256
    return pl.pallas_call(
        _stage1_body,
        grid=(BATCH // bb,),
        in_specs=[
            pl.BlockSpec((bb, W_INT), lambda i: (i, 0)),
            pl.BlockSpec((bb, 1), lambda i: (i, 0)),
        ],
        out_specs=pl.BlockSpec((bb, W_PAD), lambda i: (i, 0)),
        out_shape=jax.ShapeDtypeStruct((BATCH, W_PAD), jnp.float32),
    )(x, conc2d)


_SC_MESH = plsc.VectorSubcoreMesh(
    core_axis_name="c", subcore_axis_name="s", num_cores=NC, num_subcores=NS
)


@functools.partial(
    pl.kernel,
    mesh=_SC_MESH,
    out_type=jax.ShapeDtypeStruct((BATCH, N_RECEPTORS), jnp.float32),
    compiler_params=pltpu.CompilerParams(needs_layout_passes=False),
    scratch_types=[
        pltpu.VMEM((N_RECEPTORS * K_SUB,), jnp.int32),
        pltpu.VMEM((B_HALF, W_PAD), jnp.float32),
        pltpu.VMEM((B_HALF, N_RECEPTORS), jnp.float32),
    ],
)
def _sc_gather_sigmoid(a_hbm, idx_hbm, out_hbm, idx_v, a_v, out_v):
    wid = lax.axis_index("s") * NC + lax.axis_index("c")
    pltpu.sync_copy(idx_hbm, idx_v)
    iota5 = lax.iota(jnp.int32, LANES) * K_SUB

    for half in range(2):
        base = wid * B_PER_W + half * B_HALF
        pltpu.sync_copy(a_hbm.at[pl.ds(base, B_HALF)], a_v)

        @pl.loop(0, CHUNKS)
        def _chunk_loop(ch):
            off = ch * LANES
            iks = [
                2 * plsc.load_gather(idx_v, [iota5 + (off * K_SUB + k)])
                for k in range(K_SUB)
            ]

            res = []
            for j in range(B_HALF):
                jv = jnp.full((LANES,), j, dtype=jnp.int32)
                g = [plsc.load_gather(a_v, [jv, iks[k]]) for k in range(K_SUB)]
                s = (g[0] + g[1]) + (g[2] + g[3]) + g[4]
                res.append(1.0 / (1.0 + jnp.exp(s)))
            for j in range(B_HALF):
                out_v[j, pl.ds(off, LANES)] = res[j]

        pltpu.sync_copy(out_v, out_hbm.at[pl.ds(wid * B_PER_W + half * B_HALF, B_HALF)])


def kernel(energies, concentrations, receptor_indices):
    x = energies.reshape(BATCH, W_INT)
    conc2d = concentrations.reshape(BATCH, 1)
    a = _stage1(x, conc2d)
    idx_flat = receptor_indices.astype(jnp.int32).reshape(-1)
    return _sc_gather_sigmoid(a, idx_flat)


# speedup vs baseline: 1.1183x; 1.1164x over previous
import functools

import jax
import jax.numpy as jnp
from jax import lax
from jax.experimental import pallas as pl
from jax.experimental.pallas import tpu as pltpu
from jax.experimental.pallas import tpu_sc as plsc

N_UNITS = 1000
K_SUB = 5
BATCH = 1024
N_RECEPTORS = 4096

NC = 2
NS = 16
NW = NC * NS
LANES = 16
B_PER_W = BATCH // NW
B_RES = 8
N_PASS = B_PER_W // B_RES
CHUNKS = N_RECEPTORS // LANES
W_INT = 2 * N_UNITS
W_PAD = 2048


def _stage1_body(x_ref, c_ref, a_ref):
    x = x_ref[...]
    lnc = jnp.log(c_ref[...] + 1e-12)
    even = (
        lax.broadcasted_iota(jnp.int32, x.shape, 1) % 2
    ) == 0
    xm = lnc - x
    sp = jnp.maximum(xm, 0.0) + jnp.log1p(jnp.exp(-jnp.abs(xm)))
    g = jnp.where(even, x - lnc, sp)
    a_ref[:, :W_INT] = g + jnp.roll(g, -1, axis=1)


def _stage1(x, conc2d):
    bb = 256
    return pl.pallas_call(
        _stage1_body,
        grid=(BATCH // bb,),
        in_specs=[
            pl.BlockSpec((bb, W_INT), lambda i: (i, 0)),
            pl.BlockSpec((bb, 1), lambda i: (i, 0)),
        ],
        out_specs=pl.BlockSpec((bb, W_PAD), lambda i: (i, 0)),
        out_shape=jax.ShapeDtypeStruct((BATCH, W_PAD), jnp.float32),
    )(x, conc2d)


_SC_MESH = plsc.VectorSubcoreMesh(
    core_axis_name="c", subcore_axis_name="s", num_cores=NC, num_subcores=NS
)


@functools.partial(
    pl.kernel,
    mesh=_SC_MESH,
    out_type=jax.ShapeDtypeStruct((BATCH, N_RECEPTORS), jnp.float32),
    compiler_params=pltpu.CompilerParams(needs_layout_passes=False),
    scratch_types=[
        pltpu.VMEM((K_SUB, N_RECEPTORS), jnp.int32),
        pltpu.VMEM((B_RES, W_PAD), jnp.float32),
        pltpu.VMEM((B_RES, N_RECEPTORS), jnp.float32),
    ],
)
def _sc_gather_sigmoid(a_hbm, idx_hbm, out_hbm, idx_v, a_v, out_v):
    wid = lax.axis_index("s") * NC + lax.axis_index("c")
    pltpu.sync_copy(idx_hbm, idx_v)

    for p in range(N_PASS):
        base = wid * B_PER_W + p * B_RES
        pltpu.sync_copy(a_hbm.at[pl.ds(base, B_RES)], a_v)

        @pl.loop(0, CHUNKS)
        def _chunk_loop(ch):
            off = ch * LANES
            iks = [2 * idx_v[k, pl.ds(off, LANES)] for k in range(K_SUB)]

            res = []
            for j in range(B_RES):
                jv = jnp.full((LANES,), j, dtype=jnp.int32)
                g = [plsc.load_gather(a_v, [jv, iks[k]]) for k in range(K_SUB)]
                res.append((g[0] + g[1]) + (g[2] + g[3]) + g[4])
            for j in range(B_RES):
                out_v[j, pl.ds(off, LANES)] = res[j]

        pltpu.sync_copy(out_v, out_hbm.at[pl.ds(base, B_RES)])


def _stage3_body(s_ref, o_ref):
    o_ref[...] = 1.0 / (1.0 + jnp.exp(s_ref[...]))


def _stage3(s):
    bb = 256
    return pl.pallas_call(
        _stage3_body,
        grid=(BATCH // bb,),
        in_specs=[pl.BlockSpec((bb, N_RECEPTORS), lambda i: (i, 0))],
        out_specs=pl.BlockSpec((bb, N_RECEPTORS), lambda i: (i, 0)),
        out_shape=jax.ShapeDtypeStruct((BATCH, N_RECEPTORS), jnp.float32),
    )(s)


def kernel(energies, concentrations, receptor_indices):
    x = energies.reshape(BATCH, W_INT)
    conc2d = concentrations.reshape(BATCH, 1)
    a = _stage1(x, conc2d)
    idx_t = receptor_indices.astype(jnp.int32).T
    return _stage3(_sc_gather_sigmoid(a, idx_t))


# speedup vs baseline: 1.2499x; 1.1176x over previous
import functools

import jax
import jax.numpy as jnp
from jax import lax
from jax.experimental import pallas as pl
from jax.experimental.pallas import tpu as pltpu
from jax.experimental.pallas import tpu_sc as plsc

N_UNITS = 1000
K_SUB = 5
BATCH = 1024
N_RECEPTORS = 4096

NC = 2
NS = 16
NW = NC * NS
LANES = 16
B_PER_W = BATCH // NW
B_RES = 8
N_PASS = B_PER_W // B_RES
CHUNKS = N_RECEPTORS // LANES
W_PAD = 1024


def _stage1_body(e0_ref, e1_ref, c_ref, a_ref):
    lnc = jnp.log(c_ref[...] + 1e-12)
    x = lnc - e1_ref[...]
    sp = jnp.maximum(x, 0.0) + jnp.log1p(jnp.exp(-jnp.abs(x)))
    a_ref[:, :N_UNITS] = e0_ref[...] - lnc + sp


def _stage1(e0, e1, conc2d):
    bb = 256
    return pl.pallas_call(
        _stage1_body,
        grid=(BATCH // bb,),
        in_specs=[
            pl.BlockSpec((bb, N_UNITS), lambda i: (i, 0)),
            pl.BlockSpec((bb, N_UNITS), lambda i: (i, 0)),
            pl.BlockSpec((bb, 1), lambda i: (i, 0)),
        ],
        out_specs=pl.BlockSpec((bb, W_PAD), lambda i: (i, 0)),
        out_shape=jax.ShapeDtypeStruct((BATCH, W_PAD), jnp.float32),
    )(e0, e1, conc2d)


_SC_MESH = plsc.VectorSubcoreMesh(
    core_axis_name="c", subcore_axis_name="s", num_cores=NC, num_subcores=NS
)


@functools.partial(
    pl.kernel,
    mesh=_SC_MESH,
    out_type=jax.ShapeDtypeStruct((BATCH, N_RECEPTORS), jnp.float32),
    compiler_params=pltpu.CompilerParams(needs_layout_passes=False),
    scratch_types=[
        pltpu.VMEM((K_SUB, N_RECEPTORS), jnp.int32),
        pltpu.VMEM((B_RES, W_PAD), jnp.float32),
        pltpu.VMEM((B_RES, W_PAD), jnp.float32),
        pltpu.VMEM((B_RES, N_RECEPTORS), jnp.float32),
        pltpu.VMEM((B_RES, N_RECEPTORS), jnp.float32),
        pltpu.SemaphoreType.DMA,
        pltpu.SemaphoreType.DMA,
        pltpu.SemaphoreType.DMA,
        pltpu.SemaphoreType.DMA,
        pltpu.SemaphoreType.DMA,
    ],
)
def _sc_gather_sum(
    a_hbm, idx_hbm, out_hbm,
    idx_v, a0_v, a1_v, o0_v, o1_v,
    sem_i, sem_a0, sem_a1, sem_o0, sem_o1,
):
    wid = lax.axis_index("s") * NC + lax.axis_index("c")
    row0 = wid * B_PER_W
    a_bufs = [a0_v, a1_v]
    o_bufs = [o0_v, o1_v]
    a_sems = [sem_a0, sem_a1]
    o_sems = [sem_o0, sem_o1]

    idx_cp = pltpu.async_copy(idx_hbm, idx_v, sem_i)
    a_cp = [None, None]
    o_cp = [None, None]
    a_cp[0] = pltpu.async_copy(a_hbm.at[pl.ds(row0, B_RES)], a_bufs[0], a_sems[0])
    idx_cp.wait()

    for p in range(N_PASS):
        buf = p % 2
        base = row0 + p * B_RES
        a_cp[buf].wait()
        if p + 1 < N_PASS:
            a_cp[1 - buf] = pltpu.async_copy(
                a_hbm.at[pl.ds(base + B_RES, B_RES)], a_bufs[1 - buf], a_sems[1 - buf]
            )
        if o_cp[buf] is not None:
            o_cp[buf].wait()
        a_v = a_bufs[buf]
        out_v = o_bufs[buf]

        @pl.loop(0, CHUNKS)
        def _chunk_loop(ch):
            off = ch * LANES
            iks = [idx_v[k, pl.ds(off, LANES)] for k in range(K_SUB)]

            res = []
            for j in range(B_RES):
                jv = jnp.full((LANES,), j, dtype=jnp.int32)
                g = [plsc.load_gather(a_v, [jv, iks[k]]) for k in range(K_SUB)]
                res.append((g[0] + g[1]) + (g[2] + g[3]) + g[4])
            for j in range(B_RES):
                out_v[j, pl.ds(off, LANES)] = res[j]

        o_cp[buf] = pltpu.async_copy(out_v, out_hbm.at[pl.ds(base, B_RES)], o_sems[buf])

    for buf in range(2):
        if o_cp[buf] is not None:
            o_cp[buf].wait()


def _stage3_body(s_ref, o_ref):
    o_ref[...] = 1.0 / (1.0 + jnp.exp(s_ref[...]))


def _stage3(s):
    bb = 256
    return pl.pallas_call(
        _stage3_body,
        grid=(BATCH // bb,),
        in_specs=[pl.BlockSpec((bb, N_RECEPTORS), lambda i: (i, 0))],
        out_specs=pl.BlockSpec((bb, N_RECEPTORS), lambda i: (i, 0)),
        out_shape=jax.ShapeDtypeStruct((BATCH, N_RECEPTORS), jnp.float32),
    )(s)


def kernel(energies, concentrations, receptor_indices):
    e0 = energies[:, :, 0]
    e1 = energies[:, :, 1]
    conc2d = concentrations.reshape(BATCH, 1)
    a = _stage1(e0, e1, conc2d)
    idx_t = receptor_indices.astype(jnp.int32).T
    return _stage3(_sc_gather_sum(a, idx_t))


# speedup vs baseline: 1.2521x; 1.0018x over previous
import functools

import jax
import jax.numpy as jnp
from jax import lax
from jax.experimental import pallas as pl
from jax.experimental.pallas import tpu as pltpu
from jax.experimental.pallas import tpu_sc as plsc

N_UNITS = 1000
K_SUB = 5
BATCH = 1024
N_RECEPTORS = 4096

NC = 2
NS = 16
NW = NC * NS
LANES = 16
B_PER_W = BATCH // NW
B_RES = 8
N_PASS = B_PER_W // B_RES
CHUNKS = N_RECEPTORS // LANES
W_PAD = 1024


def _stage1_body(e0_ref, e1_ref, c_ref, a_ref):
    lnc = jnp.log(c_ref[...] + 1e-12)
    x = lnc - e1_ref[...]
    sp = jnp.maximum(x, 0.0) + jnp.log1p(jnp.exp(-jnp.abs(x)))
    a_ref[:, :N_UNITS] = e0_ref[...] - lnc + sp


def _stage1(e0, e1, conc2d):
    bb = 256
    return pl.pallas_call(
        _stage1_body,
        grid=(BATCH // bb,),
        in_specs=[
            pl.BlockSpec((bb, N_UNITS), lambda i: (i, 0)),
            pl.BlockSpec((bb, N_UNITS), lambda i: (i, 0)),
            pl.BlockSpec((bb, 1), lambda i: (i, 0)),
        ],
        out_specs=pl.BlockSpec((bb, W_PAD), lambda i: (i, 0)),
        out_shape=jax.ShapeDtypeStruct((BATCH, W_PAD), jnp.float32),
    )(e0, e1, conc2d)


_SC_MESH = plsc.VectorSubcoreMesh(
    core_axis_name="c", subcore_axis_name="s", num_cores=NC, num_subcores=NS
)


@functools.partial(
    pl.kernel,
    mesh=_SC_MESH,
    out_type=jax.ShapeDtypeStruct((BATCH, N_RECEPTORS), jnp.float32),
    compiler_params=pltpu.CompilerParams(needs_layout_passes=False),
    scratch_types=[
        pltpu.VMEM((K_SUB, N_RECEPTORS), jnp.int32),
        pltpu.VMEM((B_RES, W_PAD), jnp.float32),
        pltpu.VMEM((B_RES, W_PAD), jnp.float32),
        pltpu.VMEM((B_RES, N_RECEPTORS), jnp.float32),
        pltpu.VMEM((B_RES, N_RECEPTORS), jnp.float32),
        pltpu.SemaphoreType.DMA,
        pltpu.SemaphoreType.DMA,
        pltpu.SemaphoreType.DMA,
        pltpu.SemaphoreType.DMA,
        pltpu.SemaphoreType.DMA,
    ],
)
def _sc_gather_sum(
    a_hbm, idx_hbm, out_hbm,
    idx_v, a0_v, a1_v, o0_v, o1_v,
    sem_i, sem_a0, sem_a1, sem_o0, sem_o1,
):
    wid = lax.axis_index("s") * NC + lax.axis_index("c")
    row0 = wid * B_PER_W
    a_bufs = [a0_v, a1_v]
    o_bufs = [o0_v, o1_v]
    a_sems = [sem_a0, sem_a1]
    o_sems = [sem_o0, sem_o1]

    idx_cp = pltpu.async_copy(idx_hbm, idx_v, sem_i)
    a_cp = [None, None]
    o_cp = [None, None]
    a_cp[0] = pltpu.async_copy(a_hbm.at[pl.ds(row0, B_RES)], a_bufs[0], a_sems[0])
    idx_cp.wait()

    for p in range(N_PASS):
        buf = p % 2
        base = row0 + p * B_RES
        a_cp[buf].wait()
        if p + 1 < N_PASS:
            a_cp[1 - buf] = pltpu.async_copy(
                a_hbm.at[pl.ds(base + B_RES, B_RES)], a_bufs[1 - buf], a_sems[1 - buf]
            )
        if o_cp[buf] is not None:
            o_cp[buf].wait()
        a_v = a_bufs[buf]
        out_v = o_bufs[buf]

        @pl.loop(0, CHUNKS)
        def _chunk_loop(ch):
            off = ch * LANES
            iks = [idx_v[k, pl.ds(off, LANES)] for k in range(K_SUB)]

            res = []
            for j in range(B_RES):
                jv = jnp.full((LANES,), j, dtype=jnp.int32)
                g = [plsc.load_gather(a_v, [jv, iks[k]]) for k in range(K_SUB)]
                s = (g[0] + g[1]) + (g[2] + g[3]) + g[4]
                res.append(1.0 / (1.0 + jnp.exp(s)))
            for j in range(B_RES):
                out_v[j, pl.ds(off, LANES)] = res[j]

        o_cp[buf] = pltpu.async_copy(out_v, out_hbm.at[pl.ds(base, B_RES)], o_sems[buf])

    for buf in range(2):
        if o_cp[buf] is not None:
            o_cp[buf].wait()


def _stage3_body(s_ref, o_ref):
    o_ref[...] = 1.0 / (1.0 + jnp.exp(s_ref[...]))


def _stage3(s):
    bb = 256
    return pl.pallas_call(
        _stage3_body,
        grid=(BATCH // bb,),
        in_specs=[pl.BlockSpec((bb, N_RECEPTORS), lambda i: (i, 0))],
        out_specs=pl.BlockSpec((bb, N_RECEPTORS), lambda i: (i, 0)),
        out_shape=jax.ShapeDtypeStruct((BATCH, N_RECEPTORS), jnp.float32),
    )(s)


def kernel(energies, concentrations, receptor_indices):
    e0 = energies[:, :, 0]
    e1 = energies[:, :, 1]
    conc2d = concentrations.reshape(BATCH, 1)
    a = _stage1(e0, e1, conc2d)
    idx_t = receptor_indices.astype(jnp.int32).T
    return _sc_gather_sum(a, idx_t)
